# baseline (speedup 1.0000x reference)
import jax
import jax.numpy as jnp
from jax import lax
from jax.experimental import pallas as pl
from jax.experimental.pallas import tpu as pltpu
from jax.experimental.pallas import tpu_sc as plsc

_N = 10000
_E = 160000
_F = 128
_NC = 2
_NS = 16
_ACC = 10240
_TRASH = 10016

_DEG_CHUNK = 5008
_DEG_TOT = _DEG_CHUNK * _NC * _NS

_KJ = 84
_PE_TOT = _NS * _KJ * 128

_ROWS_PER_TILE = _N // _NS
_INIT_CHUNK = 125

_sc_mesh = plsc.VectorSubcoreMesh(
    core_axis_name="c", subcore_axis_name="s", num_cores=_NC, num_subcores=_NS
)


def _deg_body(dst_hbm, deg_out, idx_v, hist_v):
    cid = lax.axis_index("c")
    sid = lax.axis_index("s")
    wid = sid * _NC + cid

    zeros16 = jnp.zeros((16,), jnp.float32)

    def zbody(i, carry):
        hist_v[pl.ds(i * 16, 16)] = zeros16
        return carry

    lax.fori_loop(0, _ACC // 16, zbody, 0)

    pltpu.sync_copy(dst_hbm.at[pl.ds(wid * _DEG_CHUNK, _DEG_CHUNK)], idx_v)
    ones16 = jnp.full((16,), 1.0, jnp.float32)

    def sbody(k, carry):
        idx = idx_v[pl.ds(k * 16, 16)]
        plsc.addupdate_scatter(hist_v, [idx], ones16)
        return carry

    lax.fori_loop(0, _DEG_CHUNK // 16, sbody, 0)

    pltpu.sync_copy(hist_v, deg_out.at[wid])


_deg_kernel = pl.kernel(
    _deg_body,
    out_type=jax.ShapeDtypeStruct((_NC * _NS, _ACC), jnp.float32),
    mesh=_sc_mesh,
    scratch_types=[
        pltpu.VMEM((_DEG_CHUNK,), jnp.int32),
        pltpu.VMEM((_ACC,), jnp.float32),
    ],
    compiler_params=pltpu.CompilerParams(
        needs_layout_passes=False, use_tc_tiling_on_sc=False),
)


def _prop_body(h0, h1, src_hbm, dst_hbm, out0, out1, src_v, dst_v,
               rows0, rows1, acc, gsem0, gsem1, ssem0, ssem1):
    cid = lax.axis_index("c")
    sid = lax.axis_index("s")
    base = sid * _ROWS_PER_TILE

    def impl(h_hbm, out_hbm):
        for c in range(_ROWS_PER_TILE // _INIT_CHUNK):
            r0 = base + c * _INIT_CHUNK
            pltpu.sync_copy(h_hbm.at[pl.ds(r0, _INIT_CHUNK)], rows0.at[pl.ds(0, _INIT_CHUNK)])
            pltpu.sync_copy(rows0.at[pl.ds(0, _INIT_CHUNK)], acc.at[pl.ds(r0, _INIT_CHUNK)])
        plsc.subcore_barrier()

        half_kj = _KJ // 2
        for half in range(2):
            pltpu.sync_copy(src_hbm.at[sid, pl.ds(half * half_kj, half_kj)], src_v)
            pltpu.sync_copy(dst_hbm.at[sid, pl.ds(half * half_kj, half_kj)], dst_v)

            def pair_body(p, carry):
                j0 = 2 * p
                j1 = j0 + 1
                g0 = pltpu.async_copy(h_hbm.at[src_v.at[j0]], rows0, gsem0)

                @pl.when(p > 0)
                def _():
                    pltpu.make_async_copy(
                        rows1, acc.at[dst_v.at[j1]], ssem1).wait()

                g0.wait()
                s0 = pltpu.async_copy(
                    rows0, acc.at[dst_v.at[j0]], ssem0, add=True)
                g1 = pltpu.async_copy(h_hbm.at[src_v.at[j1]], rows1, gsem1)
                g1.wait()
                s0.wait()
                pltpu.async_copy(rows1, acc.at[dst_v.at[j1]], ssem1, add=True)
                return carry

            lax.fori_loop(0, half_kj // 2, pair_body, 0)
            pltpu.make_async_copy(
                rows1, acc.at[dst_v.at[half_kj - 1]], ssem1).wait()
        plsc.subcore_barrier()

        for c in range(_ROWS_PER_TILE // _INIT_CHUNK):
            r0 = base + c * _INIT_CHUNK
            pltpu.sync_copy(acc.at[pl.ds(r0, _INIT_CHUNK)], rows0.at[pl.ds(0, _INIT_CHUNK)])
            pltpu.sync_copy(rows0.at[pl.ds(0, _INIT_CHUNK)], out_hbm.at[pl.ds(r0, _INIT_CHUNK)])

    @pl.when(cid == 0)
    def _():
        impl(h0, out0)

    @pl.when(cid == 1)
    def _():
        impl(h1, out1)


_prop_kernel = pl.kernel(
    _prop_body,
    out_type=(
        jax.ShapeDtypeStruct((_N, _F), jnp.float32),
        jax.ShapeDtypeStruct((_N, _F), jnp.float32),
    ),
    mesh=_sc_mesh,
    scratch_types=[
        pltpu.VMEM((_KJ // 2, 128), jnp.int32),
        pltpu.VMEM((_KJ // 2, 128), jnp.int32),
        pltpu.VMEM((128, _F), jnp.float32),
        pltpu.VMEM((128, _F), jnp.float32),
        pltpu.VMEM_SHARED((_ACC, _F), jnp.float32),
        pltpu.SemaphoreType.DMA,
        pltpu.SemaphoreType.DMA,
        pltpu.SemaphoreType.DMA,
        pltpu.SemaphoreType.DMA,
    ],
    compiler_params=pltpu.CompilerParams(
        needs_layout_passes=False, use_tc_tiling_on_sc=False),
)


def _scale_split_body(deg_ref, x_ref, dinv_ref, g0_ref, g1_ref):
    deg = jnp.sum(deg_ref[...], axis=0)[: _N] + 1.0
    dv = lax.rsqrt(deg)
    dv2 = dv[:, None]
    dinv_ref[...] = dv2
    g = x_ref[...] * dv2
    g0_ref[...] = g[:, :_F]
    g1_ref[...] = g[:, _F:]


_scale_split = pl.pallas_call(
    _scale_split_body,
    out_shape=(
        jax.ShapeDtypeStruct((_N, 1), jnp.float32),
        jax.ShapeDtypeStruct((_N, _F), jnp.float32),
        jax.ShapeDtypeStruct((_N, _F), jnp.float32),
    ),
)

_BLK = 1000


def _mlp_body(s0_ref, s1_ref, dinv_ref, wm1_ref, bm1_ref,
              wl1_ref, bl1_ref, wm2_ref, wl2_ref, c0_ref, c1_ref):
    dv = dinv_ref[...]
    h0 = s0_ref[...] * dv
    h1 = s1_ref[...] * dv
    wm1 = wm1_ref[...]
    mu1 = jnp.maximum(
        jnp.dot(h0, wm1[:_F, :]) + jnp.dot(h1, wm1[_F:, :]) + bm1_ref[...], 0.0)
    wl1 = wl1_ref[...]
    lv1 = jnp.maximum(
        jnp.dot(h0, wl1[:_F, :]) + jnp.dot(h1, wl1[_F:, :]) + bl1_ref[...], 0.0)
    c0_ref[...] = jnp.dot(mu1, wm2_ref[...]) * dv
    c1_ref[...] = jnp.dot(lv1, wl2_ref[...]) * dv


_mlp = pl.pallas_call(
    _mlp_body,
    grid=(_N // _BLK,),
    in_specs=[
        pl.BlockSpec((_BLK, _F), lambda i: (i, 0)),
        pl.BlockSpec((_BLK, _F), lambda i: (i, 0)),
        pl.BlockSpec((_BLK, 1), lambda i: (i, 0)),
        pl.BlockSpec((2 * _F, 2 * _F), lambda i: (0, 0)),
        pl.BlockSpec((2 * _F,), lambda i: (0,)),
        pl.BlockSpec((2 * _F, 2 * _F), lambda i: (0, 0)),
        pl.BlockSpec((2 * _F,), lambda i: (0,)),
        pl.BlockSpec((2 * _F, _F), lambda i: (0, 0)),
        pl.BlockSpec((2 * _F, _F), lambda i: (0, 0)),
    ],
    out_specs=(
        pl.BlockSpec((_BLK, _F), lambda i: (i, 0)),
        pl.BlockSpec((_BLK, _F), lambda i: (i, 0)),
    ),
    out_shape=(
        jax.ShapeDtypeStruct((_N, _F), jnp.float32),
        jax.ShapeDtypeStruct((_N, _F), jnp.float32),
    ),
)


def _final_body(t0_ref, t1_ref, dinv_ref, bm2_ref, bl2_ref,
                eps_ref, z_ref, mu_ref, lv_ref):
    dv = dinv_ref[...]
    mu = t0_ref[...] * dv + bm2_ref[...]
    logvar = t1_ref[...] * dv + bl2_ref[...]
    std = jnp.exp(0.5 * logvar)
    mu_ref[...] = mu
    lv_ref[...] = logvar
    z_ref[...] = mu + eps_ref[...] * std


_final = pl.pallas_call(
    _final_body,
    grid=(_N // _BLK,),
    in_specs=[
        pl.BlockSpec((_BLK, _F), lambda i: (i, 0)),
        pl.BlockSpec((_BLK, _F), lambda i: (i, 0)),
        pl.BlockSpec((_BLK, 1), lambda i: (i, 0)),
        pl.BlockSpec((_F,), lambda i: (0,)),
        pl.BlockSpec((_F,), lambda i: (0,)),
        pl.BlockSpec((_BLK, _F), lambda i: (i, 0)),
    ],
    out_specs=(
        pl.BlockSpec((_BLK, _F), lambda i: (i, 0)),
        pl.BlockSpec((_BLK, _F), lambda i: (i, 0)),
        pl.BlockSpec((_BLK, _F), lambda i: (i, 0)),
    ),
    out_shape=(
        jax.ShapeDtypeStruct((_N, _F), jnp.float32),
        jax.ShapeDtypeStruct((_N, _F), jnp.float32),
        jax.ShapeDtypeStruct((_N, _F), jnp.float32),
    ),
)


def kernel(x, edge_index, W_mu1, b_mu1, W_mu2, b_mu2, W_lv1, b_lv1, W_lv2, b_lv2):
    src = edge_index[0]
    dst = edge_index[1]
    pad_deg = _N + (jnp.arange(_DEG_TOT - _E, dtype=jnp.int32) % (_ACC - _N))
    pad_p = _N + (jnp.arange(_PE_TOT - _E, dtype=jnp.int32) % (_ACC - _N))
    dst_deg = jnp.concatenate([dst, pad_deg])
    src_p = jnp.concatenate(
        [src, jnp.zeros((_PE_TOT - _E,), jnp.int32)]).reshape(_NS, _KJ, 128)
    dst_p = jnp.concatenate([dst, pad_p]).reshape(_NS, _KJ, 128)

    deg2 = _deg_kernel(dst_deg)
    dinv, g0, g1 = _scale_split(deg2, x)
    s0, s1 = _prop_kernel(g0, g1, src_p, dst_p)
    c0, c1 = _mlp(s0, s1, dinv, W_mu1, b_mu1, W_lv1, b_lv1, W_mu2, W_lv2)
    t0, t1 = _prop_kernel(c0, c1, src_p, dst_p)
    eps = jax.random.normal(jax.random.key(42), (_N, _F), jnp.float32)
    z, mu, logvar = _final(t0, t1, dinv, b_mu2, b_lv2, eps)
    return (z, mu, logvar)


# speedup vs baseline: 2.6384x; 2.6384x over previous
import jax
import jax.numpy as jnp
from jax import lax
from jax.experimental import pallas as pl
from jax.experimental.pallas import tpu as pltpu
from jax.experimental.pallas import tpu_sc as plsc

_N = 10000
_E = 160000
_F = 128
_NC = 2
_NS = 16
_ACC = 10240
_TRASH = 10016

_DEG_CHUNK = 5008
_DEG_TOT = _DEG_CHUNK * _NC * _NS

_KJ = 79
_PE_TOT = _NS * _KJ * 128

_ROWS_PER_TILE = _N // _NS
_INIT_CHUNK = 125

_sc_mesh = plsc.VectorSubcoreMesh(
    core_axis_name="c", subcore_axis_name="s", num_cores=_NC, num_subcores=_NS
)


def _deg_body(dst_hbm, deg_out, idx_v, hist_v):
    cid = lax.axis_index("c")
    sid = lax.axis_index("s")
    wid = sid * _NC + cid

    zeros16 = jnp.zeros((16,), jnp.float32)

    def zbody(i, carry):
        hist_v[pl.ds(i * 16, 16)] = zeros16
        return carry

    lax.fori_loop(0, _ACC // 16, zbody, 0)

    pltpu.sync_copy(dst_hbm.at[pl.ds(wid * _DEG_CHUNK, _DEG_CHUNK)], idx_v)
    ones16 = jnp.full((16,), 1.0, jnp.float32)

    def sbody(k, carry):
        idx = idx_v[pl.ds(k * 16, 16)]
        plsc.addupdate_scatter(hist_v, [idx], ones16)
        return carry

    lax.fori_loop(0, _DEG_CHUNK // 16, sbody, 0)

    pltpu.sync_copy(hist_v, deg_out.at[wid])


_deg_kernel = pl.kernel(
    _deg_body,
    out_type=jax.ShapeDtypeStruct((_NC * _NS, _ACC), jnp.float32),
    mesh=_sc_mesh,
    scratch_types=[
        pltpu.VMEM((_DEG_CHUNK,), jnp.int32),
        pltpu.VMEM((_ACC,), jnp.float32),
    ],
    compiler_params=pltpu.CompilerParams(
        needs_layout_passes=False, use_tc_tiling_on_sc=False),
)


def _prop_body(h0, h1, src_hbm, dst_hbm, out0, out1, src_v, dst_v,
               rows0, acc, gsem0):
    cid = lax.axis_index("c")
    sid = lax.axis_index("s")
    base = sid * _ROWS_PER_TILE

    def impl(h_hbm, out_hbm):
        for c in range(_ROWS_PER_TILE // _INIT_CHUNK):
            r0 = base + c * _INIT_CHUNK
            pltpu.sync_copy(h_hbm.at[pl.ds(r0, _INIT_CHUNK)], rows0.at[pl.ds(0, _INIT_CHUNK)])
            pltpu.sync_copy(rows0.at[pl.ds(0, _INIT_CHUNK)], acc.at[pl.ds(r0, _INIT_CHUNK)])
        plsc.subcore_barrier()

        pltpu.sync_copy(src_hbm.at[sid], src_v)
        pltpu.sync_copy(dst_hbm.at[sid], dst_v)

        def ebody(j, carry):
            pltpu.async_copy(h_hbm.at[src_v.at[j]], rows0, gsem0).wait()
            pltpu.sync_copy(rows0, acc.at[dst_v.at[j]], add=True)
            return carry

        lax.fori_loop(0, _KJ, ebody, 0)
        plsc.subcore_barrier()

        for c in range(_ROWS_PER_TILE // _INIT_CHUNK):
            r0 = base + c * _INIT_CHUNK
            pltpu.sync_copy(acc.at[pl.ds(r0, _INIT_CHUNK)], rows0.at[pl.ds(0, _INIT_CHUNK)])
            pltpu.sync_copy(rows0.at[pl.ds(0, _INIT_CHUNK)], out_hbm.at[pl.ds(r0, _INIT_CHUNK)])

    @pl.when(cid == 0)
    def _():
        impl(h0, out0)

    @pl.when(cid == 1)
    def _():
        impl(h1, out1)


_prop_kernel = pl.kernel(
    _prop_body,
    out_type=(
        jax.ShapeDtypeStruct((_N, _F), jnp.float32),
        jax.ShapeDtypeStruct((_N, _F), jnp.float32),
    ),
    mesh=_sc_mesh,
    scratch_types=[
        pltpu.VMEM((_KJ, 128), jnp.int32),
        pltpu.VMEM((_KJ, 128), jnp.int32),
        pltpu.VMEM((128, _F), jnp.float32),
        pltpu.VMEM_SHARED((_ACC, _F), jnp.float32),
        pltpu.SemaphoreType.DMA,
    ],
    compiler_params=pltpu.CompilerParams(
        needs_layout_passes=False, use_tc_tiling_on_sc=False),
)


def _scale_split_body(deg_ref, x_ref, dinv_ref, g0_ref, g1_ref):
    deg = jnp.sum(deg_ref[...], axis=0)[: _N] + 1.0
    dv = lax.rsqrt(deg)
    dv2 = dv[:, None]
    dinv_ref[...] = dv2
    g = x_ref[...] * dv2
    g0_ref[...] = g[:, :_F]
    g1_ref[...] = g[:, _F:]


_scale_split = pl.pallas_call(
    _scale_split_body,
    out_shape=(
        jax.ShapeDtypeStruct((_N, 1), jnp.float32),
        jax.ShapeDtypeStruct((_N, _F), jnp.float32),
        jax.ShapeDtypeStruct((_N, _F), jnp.float32),
    ),
)

_BLK = 1000


def _mlp_body(s0_ref, s1_ref, dinv_ref, wm1_ref, bm1_ref,
              wl1_ref, bl1_ref, wm2_ref, wl2_ref, c0_ref, c1_ref):
    dv = dinv_ref[...]
    h0 = s0_ref[...] * dv
    h1 = s1_ref[...] * dv
    wm1 = wm1_ref[...]
    mu1 = jnp.maximum(
        jnp.dot(h0, wm1[:_F, :]) + jnp.dot(h1, wm1[_F:, :]) + bm1_ref[...], 0.0)
    wl1 = wl1_ref[...]
    lv1 = jnp.maximum(
        jnp.dot(h0, wl1[:_F, :]) + jnp.dot(h1, wl1[_F:, :]) + bl1_ref[...], 0.0)
    c0_ref[...] = jnp.dot(mu1, wm2_ref[...]) * dv
    c1_ref[...] = jnp.dot(lv1, wl2_ref[...]) * dv


_mlp = pl.pallas_call(
    _mlp_body,
    grid=(_N // _BLK,),
    in_specs=[
        pl.BlockSpec((_BLK, _F), lambda i: (i, 0)),
        pl.BlockSpec((_BLK, _F), lambda i: (i, 0)),
        pl.BlockSpec((_BLK, 1), lambda i: (i, 0)),
        pl.BlockSpec((2 * _F, 2 * _F), lambda i: (0, 0)),
        pl.BlockSpec((2 * _F,), lambda i: (0,)),
        pl.BlockSpec((2 * _F, 2 * _F), lambda i: (0, 0)),
        pl.BlockSpec((2 * _F,), lambda i: (0,)),
        pl.BlockSpec((2 * _F, _F), lambda i: (0, 0)),
        pl.BlockSpec((2 * _F, _F), lambda i: (0, 0)),
    ],
    out_specs=(
        pl.BlockSpec((_BLK, _F), lambda i: (i, 0)),
        pl.BlockSpec((_BLK, _F), lambda i: (i, 0)),
    ),
    out_shape=(
        jax.ShapeDtypeStruct((_N, _F), jnp.float32),
        jax.ShapeDtypeStruct((_N, _F), jnp.float32),
    ),
)


def _final_body(t0_ref, t1_ref, dinv_ref, bm2_ref, bl2_ref,
                eps_ref, z_ref, mu_ref, lv_ref):
    dv = dinv_ref[...]
    mu = t0_ref[...] * dv + bm2_ref[...]
    logvar = t1_ref[...] * dv + bl2_ref[...]
    std = jnp.exp(0.5 * logvar)
    mu_ref[...] = mu
    lv_ref[...] = logvar
    z_ref[...] = mu + eps_ref[...] * std


_final = pl.pallas_call(
    _final_body,
    grid=(_N // _BLK,),
    in_specs=[
        pl.BlockSpec((_BLK, _F), lambda i: (i, 0)),
        pl.BlockSpec((_BLK, _F), lambda i: (i, 0)),
        pl.BlockSpec((_BLK, 1), lambda i: (i, 0)),
        pl.BlockSpec((_F,), lambda i: (0,)),
        pl.BlockSpec((_F,), lambda i: (0,)),
        pl.BlockSpec((_BLK, _F), lambda i: (i, 0)),
    ],
    out_specs=(
        pl.BlockSpec((_BLK, _F), lambda i: (i, 0)),
        pl.BlockSpec((_BLK, _F), lambda i: (i, 0)),
        pl.BlockSpec((_BLK, _F), lambda i: (i, 0)),
    ),
    out_shape=(
        jax.ShapeDtypeStruct((_N, _F), jnp.float32),
        jax.ShapeDtypeStruct((_N, _F), jnp.float32),
        jax.ShapeDtypeStruct((_N, _F), jnp.float32),
    ),
)


def kernel(x, edge_index, W_mu1, b_mu1, W_mu2, b_mu2, W_lv1, b_lv1, W_lv2, b_lv2):
    src = edge_index[0]
    dst = edge_index[1]
    pad_deg = _N + (jnp.arange(_DEG_TOT - _E, dtype=jnp.int32) % (_ACC - _N))
    pad_p = _N + (jnp.arange(_PE_TOT - _E, dtype=jnp.int32) % (_ACC - _N))
    dst_deg = jnp.concatenate([dst, pad_deg])
    src_p = jnp.concatenate(
        [src, jnp.zeros((_PE_TOT - _E,), jnp.int32)]).reshape(_NS, _KJ, 128)
    dst_p = jnp.concatenate([dst, pad_p]).reshape(_NS, _KJ, 128)

    deg2 = _deg_kernel(dst_deg)
    dinv, g0, g1 = _scale_split(deg2, x)
    s0, s1 = _prop_kernel(g0, g1, src_p, dst_p)
    c0, c1 = _mlp(s0, s1, dinv, W_mu1, b_mu1, W_lv1, b_lv1, W_mu2, W_lv2)
    t0, t1 = _prop_kernel(c0, c1, src_p, dst_p)
    eps = jax.random.normal(jax.random.key(42), (_N, _F), jnp.float32)
    z, mu, logvar = _final(t0, t1, dinv, b_mu2, b_lv2, eps)
    return (z, mu, logvar)


# speedup vs baseline: 2.7039x; 1.0248x over previous
import jax
import jax.numpy as jnp
from jax import lax
from jax.experimental import pallas as pl
from jax.experimental.pallas import tpu as pltpu
from jax.experimental.pallas import tpu_sc as plsc

_N = 10000
_E = 160000
_F = 128
_NC = 2
_NS = 16
_ACC = 10240
_TRASH = 10016

_DEG_CHUNK = 5008
_DEG_TOT = _DEG_CHUNK * _NC * _NS

_KJ = 79
_PE_TOT = _NS * _KJ * 128

_ROWS_PER_TILE = _N // _NS
_INIT_CHUNK = 125

_sc_mesh = plsc.VectorSubcoreMesh(
    core_axis_name="c", subcore_axis_name="s", num_cores=_NC, num_subcores=_NS
)


def _deg_body(dst_hbm, deg_out, idx_v, hist_v):
    cid = lax.axis_index("c")
    sid = lax.axis_index("s")
    wid = sid * _NC + cid

    zeros16 = jnp.zeros((16,), jnp.float32)

    def zbody(i, carry):
        hist_v[pl.ds(i * 16, 16)] = zeros16
        return carry

    lax.fori_loop(0, _ACC // 16, zbody, 0)

    pltpu.sync_copy(dst_hbm.at[pl.ds(wid * _DEG_CHUNK, _DEG_CHUNK)], idx_v)
    ones16 = jnp.full((16,), 1.0, jnp.float32)

    def sbody(k, carry):
        idx = idx_v[pl.ds(k * 16, 16)]
        plsc.addupdate_scatter(hist_v, [idx], ones16)
        return carry

    lax.fori_loop(0, _DEG_CHUNK // 16, sbody, 0)

    pltpu.sync_copy(hist_v, deg_out.at[wid])


_deg_kernel = pl.kernel(
    _deg_body,
    out_type=jax.ShapeDtypeStruct((_NC * _NS, _ACC), jnp.float32),
    mesh=_sc_mesh,
    scratch_types=[
        pltpu.VMEM((_DEG_CHUNK,), jnp.int32),
        pltpu.VMEM((_ACC,), jnp.float32),
    ],
    compiler_params=pltpu.CompilerParams(
        needs_layout_passes=False, use_tc_tiling_on_sc=False),
)


def _prop_body(h0, h1, src_hbm, dst_hbm, out0, out1, src_v, dst_v,
               rows0, acc, gsem0):
    cid = lax.axis_index("c")
    sid = lax.axis_index("s")
    base = sid * _ROWS_PER_TILE

    def impl(h_hbm, out_hbm):
        pltpu.sync_copy(h_hbm.at[pl.ds(base, _ROWS_PER_TILE)],
                        acc.at[pl.ds(base, _ROWS_PER_TILE)])
        plsc.subcore_barrier()

        pltpu.sync_copy(src_hbm.at[sid], src_v)
        pltpu.sync_copy(dst_hbm.at[sid], dst_v)

        def ebody(j, carry):
            pltpu.async_copy(h_hbm.at[src_v.at[j]], rows0, gsem0).wait()
            pltpu.sync_copy(rows0, acc.at[dst_v.at[j]], add=True)
            return carry

        lax.fori_loop(0, _KJ, ebody, 0)
        plsc.subcore_barrier()

        pltpu.sync_copy(acc.at[pl.ds(base, _ROWS_PER_TILE)],
                        out_hbm.at[pl.ds(base, _ROWS_PER_TILE)])

    @pl.when(cid == 0)
    def _():
        impl(h0, out0)

    @pl.when(cid == 1)
    def _():
        impl(h1, out1)


_prop_kernel = pl.kernel(
    _prop_body,
    out_type=(
        jax.ShapeDtypeStruct((_N, _F), jnp.float32),
        jax.ShapeDtypeStruct((_N, _F), jnp.float32),
    ),
    mesh=_sc_mesh,
    scratch_types=[
        pltpu.VMEM((_KJ, 128), jnp.int32),
        pltpu.VMEM((_KJ, 128), jnp.int32),
        pltpu.VMEM((128, _F), jnp.float32),
        pltpu.VMEM_SHARED((_ACC, _F), jnp.float32),
        pltpu.SemaphoreType.DMA,
    ],
    compiler_params=pltpu.CompilerParams(
        needs_layout_passes=False, use_tc_tiling_on_sc=False),
)


def _scale_split_body(deg_ref, x_ref, dinv_ref, g0_ref, g1_ref):
    deg = jnp.sum(deg_ref[...], axis=0)[: _N] + 1.0
    dv = lax.rsqrt(deg)
    dv2 = dv[:, None]
    dinv_ref[...] = dv2
    g = x_ref[...] * dv2
    g0_ref[...] = g[:, :_F]
    g1_ref[...] = g[:, _F:]


_scale_split = pl.pallas_call(
    _scale_split_body,
    out_shape=(
        jax.ShapeDtypeStruct((_N, 1), jnp.float32),
        jax.ShapeDtypeStruct((_N, _F), jnp.float32),
        jax.ShapeDtypeStruct((_N, _F), jnp.float32),
    ),
)

_BLK = 1000


def _mlp_body(s0_ref, s1_ref, dinv_ref, wm1_ref, bm1_ref,
              wl1_ref, bl1_ref, wm2_ref, wl2_ref, c0_ref, c1_ref):
    dv = dinv_ref[...]
    h0 = s0_ref[...] * dv
    h1 = s1_ref[...] * dv
    wm1 = wm1_ref[...]
    mu1 = jnp.maximum(
        jnp.dot(h0, wm1[:_F, :]) + jnp.dot(h1, wm1[_F:, :]) + bm1_ref[...], 0.0)
    wl1 = wl1_ref[...]
    lv1 = jnp.maximum(
        jnp.dot(h0, wl1[:_F, :]) + jnp.dot(h1, wl1[_F:, :]) + bl1_ref[...], 0.0)
    c0_ref[...] = jnp.dot(mu1, wm2_ref[...]) * dv
    c1_ref[...] = jnp.dot(lv1, wl2_ref[...]) * dv


_mlp = pl.pallas_call(
    _mlp_body,
    grid=(_N // _BLK,),
    in_specs=[
        pl.BlockSpec((_BLK, _F), lambda i: (i, 0)),
        pl.BlockSpec((_BLK, _F), lambda i: (i, 0)),
        pl.BlockSpec((_BLK, 1), lambda i: (i, 0)),
        pl.BlockSpec((2 * _F, 2 * _F), lambda i: (0, 0)),
        pl.BlockSpec((2 * _F,), lambda i: (0,)),
        pl.BlockSpec((2 * _F, 2 * _F), lambda i: (0, 0)),
        pl.BlockSpec((2 * _F,), lambda i: (0,)),
        pl.BlockSpec((2 * _F, _F), lambda i: (0, 0)),
        pl.BlockSpec((2 * _F, _F), lambda i: (0, 0)),
    ],
    out_specs=(
        pl.BlockSpec((_BLK, _F), lambda i: (i, 0)),
        pl.BlockSpec((_BLK, _F), lambda i: (i, 0)),
    ),
    out_shape=(
        jax.ShapeDtypeStruct((_N, _F), jnp.float32),
        jax.ShapeDtypeStruct((_N, _F), jnp.float32),
    ),
)


def _final_body(t0_ref, t1_ref, dinv_ref, bm2_ref, bl2_ref,
                eps_ref, z_ref, mu_ref, lv_ref):
    dv = dinv_ref[...]
    mu = t0_ref[...] * dv + bm2_ref[...]
    logvar = t1_ref[...] * dv + bl2_ref[...]
    std = jnp.exp(0.5 * logvar)
    mu_ref[...] = mu
    lv_ref[...] = logvar
    z_ref[...] = mu + eps_ref[...] * std


_final = pl.pallas_call(
    _final_body,
    grid=(_N // _BLK,),
    in_specs=[
        pl.BlockSpec((_BLK, _F), lambda i: (i, 0)),
        pl.BlockSpec((_BLK, _F), lambda i: (i, 0)),
        pl.BlockSpec((_BLK, 1), lambda i: (i, 0)),
        pl.BlockSpec((_F,), lambda i: (0,)),
        pl.BlockSpec((_F,), lambda i: (0,)),
        pl.BlockSpec((_BLK, _F), lambda i: (i, 0)),
    ],
    out_specs=(
        pl.BlockSpec((_BLK, _F), lambda i: (i, 0)),
        pl.BlockSpec((_BLK, _F), lambda i: (i, 0)),
        pl.BlockSpec((_BLK, _F), lambda i: (i, 0)),
    ),
    out_shape=(
        jax.ShapeDtypeStruct((_N, _F), jnp.float32),
        jax.ShapeDtypeStruct((_N, _F), jnp.float32),
        jax.ShapeDtypeStruct((_N, _F), jnp.float32),
    ),
)


def kernel(x, edge_index, W_mu1, b_mu1, W_mu2, b_mu2, W_lv1, b_lv1, W_lv2, b_lv2):
    src = edge_index[0]
    dst = edge_index[1]
    pad_deg = _N + (jnp.arange(_DEG_TOT - _E, dtype=jnp.int32) % (_ACC - _N))
    pad_p = _N + (jnp.arange(_PE_TOT - _E, dtype=jnp.int32) % (_ACC - _N))
    dst_deg = jnp.concatenate([dst, pad_deg])
    src_p = jnp.concatenate(
        [src, jnp.zeros((_PE_TOT - _E,), jnp.int32)]).reshape(_NS, _KJ, 128)
    dst_p = jnp.concatenate([dst, pad_p]).reshape(_NS, _KJ, 128)

    deg2 = _deg_kernel(dst_deg)
    dinv, g0, g1 = _scale_split(deg2, x)
    s0, s1 = _prop_kernel(g0, g1, src_p, dst_p)
    c0, c1 = _mlp(s0, s1, dinv, W_mu1, b_mu1, W_lv1, b_lv1, W_mu2, W_lv2)
    t0, t1 = _prop_kernel(c0, c1, src_p, dst_p)
    eps = jax.random.normal(jax.random.key(42), (_N, _F), jnp.float32)
    z, mu, logvar = _final(t0, t1, dinv, b_mu2, b_lv2, eps)
    return (z, mu, logvar)


# speedup vs baseline: 2.7171x; 1.0049x over previous
import jax
import jax.numpy as jnp
from jax import lax
from jax.experimental import pallas as pl
from jax.experimental.pallas import tpu as pltpu
from jax.experimental.pallas import tpu_sc as plsc

_N = 10000
_E = 160000
_F = 128
_NC = 2
_NS = 16
_ACC = 10240
_TRASH = 10016

_DEG_CHUNK = 5008
_DEG_TOT = _DEG_CHUNK * _NC * _NS

_KJ = 79
_PE_TOT = _NS * _KJ * 128

_ROWS_PER_TILE = _N // _NS
_INIT_CHUNK = 125

_sc_mesh = plsc.VectorSubcoreMesh(
    core_axis_name="c", subcore_axis_name="s", num_cores=_NC, num_subcores=_NS
)


def _deg_body(dst_hbm, deg_out, idx_v, hist_v):
    cid = lax.axis_index("c")
    sid = lax.axis_index("s")
    wid = sid * _NC + cid

    zeros16 = jnp.zeros((16,), jnp.float32)

    def zbody(i, carry):
        hist_v[pl.ds(i * 16, 16)] = zeros16
        return carry

    lax.fori_loop(0, _ACC // 16, zbody, 0)

    pltpu.sync_copy(dst_hbm.at[pl.ds(wid * _DEG_CHUNK, _DEG_CHUNK)], idx_v)
    ones16 = jnp.full((16,), 1.0, jnp.float32)

    def sbody(k, carry):
        idx = idx_v[pl.ds(k * 16, 16)]
        plsc.addupdate_scatter(hist_v, [idx], ones16)
        return carry

    lax.fori_loop(0, _DEG_CHUNK // 16, sbody, 0)

    pltpu.sync_copy(hist_v, deg_out.at[wid])


_deg_kernel = pl.kernel(
    _deg_body,
    out_type=jax.ShapeDtypeStruct((_NC * _NS, _ACC), jnp.float32),
    mesh=_sc_mesh,
    scratch_types=[
        pltpu.VMEM((_DEG_CHUNK,), jnp.int32),
        pltpu.VMEM((_ACC,), jnp.float32),
    ],
    compiler_params=pltpu.CompilerParams(
        needs_layout_passes=False, use_tc_tiling_on_sc=False),
)


def _prop_body(h0, h1, src_hbm, dst_hbm, out0, out1, src_v, dst_v,
               rows0, acc, gsem0):
    cid = lax.axis_index("c")
    sid = lax.axis_index("s")
    base = sid * _ROWS_PER_TILE

    def impl(h_hbm, out_hbm):
        i0 = pltpu.async_copy(h_hbm.at[pl.ds(base, _ROWS_PER_TILE)],
                              acc.at[pl.ds(base, _ROWS_PER_TILE)], gsem0)
        pltpu.sync_copy(src_hbm.at[sid], src_v)
        pltpu.sync_copy(dst_hbm.at[sid], dst_v)
        i0.wait()
        plsc.subcore_barrier()


        def ebody(j, carry):
            pltpu.async_copy(h_hbm.at[src_v.at[j]], rows0, gsem0).wait()
            pltpu.sync_copy(rows0, acc.at[dst_v.at[j]], add=True)
            return carry

        lax.fori_loop(0, _KJ, ebody, 0)
        plsc.subcore_barrier()

        pltpu.sync_copy(acc.at[pl.ds(base, _ROWS_PER_TILE)],
                        out_hbm.at[pl.ds(base, _ROWS_PER_TILE)])

    @pl.when(cid == 0)
    def _():
        impl(h0, out0)

    @pl.when(cid == 1)
    def _():
        impl(h1, out1)


_prop_kernel = pl.kernel(
    _prop_body,
    out_type=(
        jax.ShapeDtypeStruct((_N, _F), jnp.float32),
        jax.ShapeDtypeStruct((_N, _F), jnp.float32),
    ),
    mesh=_sc_mesh,
    scratch_types=[
        pltpu.VMEM((_KJ, 128), jnp.int32),
        pltpu.VMEM((_KJ, 128), jnp.int32),
        pltpu.VMEM((128, _F), jnp.float32),
        pltpu.VMEM_SHARED((_ACC, _F), jnp.float32),
        pltpu.SemaphoreType.DMA,
    ],
    compiler_params=pltpu.CompilerParams(
        needs_layout_passes=False, use_tc_tiling_on_sc=False),
)


def _scale_split_body(deg_ref, x_ref, dinv_ref, g0_ref, g1_ref):
    deg = jnp.sum(deg_ref[...], axis=0)[: _N] + 1.0
    dv = lax.rsqrt(deg)
    dv2 = dv[:, None]
    dinv_ref[...] = dv2
    g = x_ref[...] * dv2
    g0_ref[...] = g[:, :_F]
    g1_ref[...] = g[:, _F:]


_scale_split = pl.pallas_call(
    _scale_split_body,
    out_shape=(
        jax.ShapeDtypeStruct((_N, 1), jnp.float32),
        jax.ShapeDtypeStruct((_N, _F), jnp.float32),
        jax.ShapeDtypeStruct((_N, _F), jnp.float32),
    ),
)

_BLK = 1000


def _mlp_body(s0_ref, s1_ref, dinv_ref, wm1_ref, bm1_ref,
              wl1_ref, bl1_ref, wm2_ref, wl2_ref, c0_ref, c1_ref):
    dv = dinv_ref[...]
    h0 = s0_ref[...] * dv
    h1 = s1_ref[...] * dv
    wm1 = wm1_ref[...]
    mu1 = jnp.maximum(
        jnp.dot(h0, wm1[:_F, :]) + jnp.dot(h1, wm1[_F:, :]) + bm1_ref[...], 0.0)
    wl1 = wl1_ref[...]
    lv1 = jnp.maximum(
        jnp.dot(h0, wl1[:_F, :]) + jnp.dot(h1, wl1[_F:, :]) + bl1_ref[...], 0.0)
    c0_ref[...] = jnp.dot(mu1, wm2_ref[...]) * dv
    c1_ref[...] = jnp.dot(lv1, wl2_ref[...]) * dv


_mlp = pl.pallas_call(
    _mlp_body,
    grid=(_N // _BLK,),
    in_specs=[
        pl.BlockSpec((_BLK, _F), lambda i: (i, 0)),
        pl.BlockSpec((_BLK, _F), lambda i: (i, 0)),
        pl.BlockSpec((_BLK, 1), lambda i: (i, 0)),
        pl.BlockSpec((2 * _F, 2 * _F), lambda i: (0, 0)),
        pl.BlockSpec((2 * _F,), lambda i: (0,)),
        pl.BlockSpec((2 * _F, 2 * _F), lambda i: (0, 0)),
        pl.BlockSpec((2 * _F,), lambda i: (0,)),
        pl.BlockSpec((2 * _F, _F), lambda i: (0, 0)),
        pl.BlockSpec((2 * _F, _F), lambda i: (0, 0)),
    ],
    out_specs=(
        pl.BlockSpec((_BLK, _F), lambda i: (i, 0)),
        pl.BlockSpec((_BLK, _F), lambda i: (i, 0)),
    ),
    out_shape=(
        jax.ShapeDtypeStruct((_N, _F), jnp.float32),
        jax.ShapeDtypeStruct((_N, _F), jnp.float32),
    ),
)


def _final_body(t0_ref, t1_ref, dinv_ref, bm2_ref, bl2_ref,
                eps_ref, z_ref, mu_ref, lv_ref):
    dv = dinv_ref[...]
    mu = t0_ref[...] * dv + bm2_ref[...]
    logvar = t1_ref[...] * dv + bl2_ref[...]
    std = jnp.exp(0.5 * logvar)
    mu_ref[...] = mu
    lv_ref[...] = logvar
    z_ref[...] = mu + eps_ref[...] * std


_final = pl.pallas_call(
    _final_body,
    grid=(_N // _BLK,),
    in_specs=[
        pl.BlockSpec((_BLK, _F), lambda i: (i, 0)),
        pl.BlockSpec((_BLK, _F), lambda i: (i, 0)),
        pl.BlockSpec((_BLK, 1), lambda i: (i, 0)),
        pl.BlockSpec((_F,), lambda i: (0,)),
        pl.BlockSpec((_F,), lambda i: (0,)),
        pl.BlockSpec((_BLK, _F), lambda i: (i, 0)),
    ],
    out_specs=(
        pl.BlockSpec((_BLK, _F), lambda i: (i, 0)),
        pl.BlockSpec((_BLK, _F), lambda i: (i, 0)),
        pl.BlockSpec((_BLK, _F), lambda i: (i, 0)),
    ),
    out_shape=(
        jax.ShapeDtypeStruct((_N, _F), jnp.float32),
        jax.ShapeDtypeStruct((_N, _F), jnp.float32),
        jax.ShapeDtypeStruct((_N, _F), jnp.float32),
    ),
)


def kernel(x, edge_index, W_mu1, b_mu1, W_mu2, b_mu2, W_lv1, b_lv1, W_lv2, b_lv2):
    src = edge_index[0]
    dst = edge_index[1]
    pad_deg = _N + (jnp.arange(_DEG_TOT - _E, dtype=jnp.int32) % (_ACC - _N))
    pad_p = _N + (jnp.arange(_PE_TOT - _E, dtype=jnp.int32) % (_ACC - _N))
    dst_deg = jnp.concatenate([dst, pad_deg])
    src_p = jnp.concatenate(
        [src, jnp.zeros((_PE_TOT - _E,), jnp.int32)]).reshape(_NS, _KJ, 128)
    dst_p = jnp.concatenate([dst, pad_p]).reshape(_NS, _KJ, 128)

    deg2 = _deg_kernel(dst_deg)
    dinv, g0, g1 = _scale_split(deg2, x)
    s0, s1 = _prop_kernel(g0, g1, src_p, dst_p)
    c0, c1 = _mlp(s0, s1, dinv, W_mu1, b_mu1, W_lv1, b_lv1, W_mu2, W_lv2)
    t0, t1 = _prop_kernel(c0, c1, src_p, dst_p)
    eps = jax.random.normal(jax.random.key(42), (_N, _F), jnp.float32)
    z, mu, logvar = _final(t0, t1, dinv, b_mu2, b_lv2, eps)
    return (z, mu, logvar)
